# Initial kernel scaffold; baseline (speedup 1.0000x reference)
import jax
import jax.numpy as jnp
from jax.experimental import pallas as pl


def kernel(x, table):
    raise NotImplementedError("write your pallas kernel here")


# speedup vs baseline: 1.0940x; 1.0940x over previous
import functools

import jax
import jax.numpy as jnp
from jax import lax
from jax.experimental import pallas as pl
from jax.experimental.pallas import tpu as pltpu
from jax.experimental.pallas import tpu_sc as plsc

NC = 2
NS = 16
NW = NC * NS
RW = 128
CHUNK_ROWS = 8
CHUNK = CHUNK_ROWS * RW


def _gather_body(n_rows, emb, x_hbm, table_hbm, out_hbm, idx_v, rows_v, sem):
    wid = lax.axis_index("s") * NC + lax.axis_index("c")
    rows_per_w = n_rows // NW
    n_chunks = rows_per_w // CHUNK_ROWS
    row0 = wid * rows_per_w

    def chunk(i, carry):
        r = row0 + i * CHUNK_ROWS
        pltpu.sync_copy(x_hbm.at[pl.ds(r, CHUNK_ROWS)], idx_v)
        descs = [
            pltpu.async_copy(
                table_hbm.at[idx_v.at[j]],
                rows_v.at[pl.ds(j * RW, RW)],
                sem,
            )
            for j in range(CHUNK_ROWS)
        ]
        for d in descs:
            d.wait()
        pltpu.sync_copy(rows_v, out_hbm.at[pl.ds(r * RW, CHUNK)])
        return carry

    lax.fori_loop(0, n_chunks, chunk, 0)


def kernel(x, table):
    b, s = x.shape
    vocab, emb = table.shape
    n = b * s
    n_rows = n // RW
    x2 = x.reshape(n_rows, RW)

    embed = pl.kernel(
        functools.partial(_gather_body, n_rows, emb),
        out_type=jax.ShapeDtypeStruct((n, emb), jnp.float32),
        mesh=plsc.VectorSubcoreMesh(core_axis_name="c", subcore_axis_name="s"),
        compiler_params=pltpu.CompilerParams(use_tc_tiling_on_sc=False),
        scratch_types=[
            pltpu.VMEM((CHUNK_ROWS, RW), jnp.int32),
            pltpu.VMEM((CHUNK, emb), jnp.float32),
            pltpu.SemaphoreType.DMA,
        ],
    )
    out = embed(x2, table)
    return out.reshape(b, s, emb)


# speedup vs baseline: 1.7418x; 1.5921x over previous
import functools

import jax
import jax.numpy as jnp
from jax import lax
from jax.experimental import pallas as pl
from jax.experimental.pallas import tpu as pltpu
from jax.experimental.pallas import tpu_sc as plsc

NC = 2
NS = 16
NW = NC * NS
SEQ_CHUNK = 16


def _gather_body(n_seq, seq_len, emb, x_hbm, table_hbm, out_hbm,
                 idx_v, rows_v, sem):
    wid = lax.axis_index("s") * NC + lax.axis_index("c")
    seq_per_w = n_seq // NW
    n_chunks = seq_per_w // SEQ_CHUNK
    seq0 = wid * seq_per_w

    def chunk(i, carry):
        s = seq0 + i * SEQ_CHUNK
        pltpu.sync_copy(x_hbm.at[pl.ds(s, SEQ_CHUNK)], idx_v)
        descs = [
            pltpu.async_copy(table_hbm.at[idx_v.at[j]], rows_v.at[j], sem)
            for j in range(SEQ_CHUNK)
        ]
        for d in descs:
            d.wait()
        pltpu.sync_copy(rows_v, out_hbm.at[pl.ds(s, SEQ_CHUNK)])
        return carry

    lax.fori_loop(0, n_chunks, chunk, 0)


def kernel(x, table):
    n_seq, seq_len = x.shape
    vocab, emb = table.shape

    embed = pl.kernel(
        functools.partial(_gather_body, n_seq, seq_len, emb),
        out_type=jax.ShapeDtypeStruct((n_seq, seq_len, emb), jnp.float32),
        mesh=plsc.VectorSubcoreMesh(core_axis_name="c", subcore_axis_name="s"),
        compiler_params=pltpu.CompilerParams(use_tc_tiling_on_sc=False),
        scratch_types=[
            pltpu.VMEM((SEQ_CHUNK, seq_len), jnp.int32),
            pltpu.VMEM((SEQ_CHUNK, seq_len, emb), jnp.float32),
            pltpu.SemaphoreType.DMA,
        ],
    )
    return embed(x, table)
